# baseline (speedup 1.0000x reference)
import functools

import jax
import jax.numpy as jnp
from jax import lax
from jax.experimental import pallas as pl
from jax.experimental.pallas import tpu as pltpu
from jax.experimental.pallas import tpu_sc as plsc

N = 10000
E = 320000
IN_DIM = 128
DIM = 128
NUM_HEAD = 8
HEAD_DIM = DIM // NUM_HEAD
SCALE = 1.0 / DIM ** 0.5

NC = 2
NS = 16
HGRP = 4
HCOLS = HGRP * HEAD_DIM

EPS = E // NS
CP = 16
SEGE = 4000
NSEG = EPS // SEGE
NCHS = SEGE // CP
PAIRS = NCHS // 2
RPT = 624
ZROWS = 16
TAIL = N - NS * RPT


def _proj_body(hc_ref, hp_ref, wqc_ref, wqp_ref, wk_ref, wv_ref,
               qt_ref, kvt_ref):
    hc = hc_ref[...]
    hp = hp_ref[...]
    qc = jnp.dot(hc, wqc_ref[...], preferred_element_type=jnp.float32) * SCALE
    qp = jnp.dot(hp, wqp_ref[...], preferred_element_type=jnp.float32) * SCALE
    k = jnp.dot(hc, wk_ref[...], preferred_element_type=jnp.float32)
    v = jnp.dot(hc, wv_ref[...], preferred_element_type=jnp.float32)
    qt_ref[0] = jnp.concatenate([qc[:, :HCOLS], qp[:, :HCOLS]], axis=1)
    qt_ref[1] = jnp.concatenate([qc[:, HCOLS:], qp[:, HCOLS:]], axis=1)
    kvt_ref[0] = jnp.concatenate([k[:, :HCOLS], v[:, :HCOLS]], axis=1)
    kvt_ref[1] = jnp.concatenate([k[:, HCOLS:], v[:, HCOLS:]], axis=1)


def _project(h_c, h_p, Wq_c, Wq_p, Wk, Wv):
    R = 1000
    grid = (N // R,)
    in_specs = [
        pl.BlockSpec((R, IN_DIM), lambda i: (i, 0)),
        pl.BlockSpec((R, IN_DIM), lambda i: (i, 0)),
    ] + [pl.BlockSpec((IN_DIM, DIM), lambda i: (0, 0))] * 4
    out_specs = [pl.BlockSpec((NC, R, DIM), lambda i: (0, i, 0))] * 2
    out_shape = [jax.ShapeDtypeStruct((NC, N, DIM), jnp.float32)] * 2
    return pl.pallas_call(
        _proj_body,
        grid=grid,
        in_specs=in_specs,
        out_specs=out_specs,
        out_shape=out_shape,
    )(h_c, h_p, Wq_c, Wq_p, Wk, Wv)


def _edge_body(src_ref, dst_ref, qt_ref, kvt_ref, out_ref,
               src_seg, dst_seg,
               qva, kvva, qvb, kvvb, ova, ovb, zb, acc,
               semga, semgb, semsa, semsb):
    cid = lax.axis_index("c")
    sid = lax.axis_index("s")

    zeros16 = jnp.zeros((16,), jnp.float32)

    def zrow(r, carry):
        for j in range(DIM // 16):
            zb[r, pl.ds(j * 16, 16)] = zeros16
        return carry

    lax.fori_loop(0, ZROWS, zrow, 0)
    row0 = sid * RPT
    for j in range(RPT // ZROWS):
        pltpu.sync_copy(zb, acc.at[pl.ds(row0 + j * ZROWS, ZROWS)])

    @pl.when(sid == NS - 1)
    def _zero_tail():
        pltpu.sync_copy(zb.at[pl.ds(0, TAIL)], acc.at[pl.ds(NS * RPT, TAIL)])

    plsc.subcore_barrier()

    qt = qt_ref.at[cid]
    kvt = kvt_ref.at[cid]

    rows = lax.iota(jnp.int32, 16)

    def idxvecs(ci):
        sv = src_seg[pl.ds(ci * CP, CP)]
        dv = dst_seg[pl.ds(ci * CP, CP)]
        return sv, dv

    def start_gathers(ci, kvb, qb, sem):
        sv, dv = idxvecs(ci)
        pltpu.async_copy(kvt.at[sv], kvb, sem)
        pltpu.async_copy(qt.at[dv], qb, sem)

    def wait_gathers(ci, kvb, qb, sem):
        sv, dv = idxvecs(ci)
        pltpu.make_async_copy(kvt.at[sv], kvb, sem).wait()
        pltpu.make_async_copy(qt.at[dv], qb, sem).wait()

    def compute(qb, kvb, ov):
        for h in range(HGRP):
            colv = jnp.full((16,), h * HEAD_DIM, jnp.int32)
            pc = []
            pp = []
            for d in range(HEAD_DIM):
                cv = colv + d
                cv64 = cv + HCOLS
                kc = plsc.load_gather(kvb, [rows, cv])
                qcc = plsc.load_gather(qb, [rows, cv])
                qpc = plsc.load_gather(qb, [rows, cv64])
                pc.append(qcc * kc)
                pp.append(qpc * kc)
            while len(pc) > 1:
                pc = [pc[i] + pc[i + 1] for i in range(0, len(pc), 2)]
                pp = [pp[i] + pp[i + 1] for i in range(0, len(pp), 2)]
            wc = jnp.exp(jnp.clip(pc[0], -5.0, 5.0))
            wp = jnp.exp(jnp.clip(pp[0], -5.0, 5.0))
            for d in range(HEAD_DIM):
                cv = colv + d
                cv64 = cv + HCOLS
                vc = plsc.load_gather(kvb, [rows, cv64])
                plsc.store_scatter(ov, [rows, cv], vc * wc)
                plsc.store_scatter(ov, [rows, cv64], vc * wp)

    def seg_body(sg, carry):
        segbase = sid * EPS + sg * SEGE
        pltpu.sync_copy(src_ref.at[pl.ds(segbase, SEGE)], src_seg)
        pltpu.sync_copy(dst_ref.at[pl.ds(segbase, SEGE)], dst_seg)

        start_gathers(0, kvva, qva, semga)

        def pair(i2, c2):
            n = 2 * i2
            start_gathers(n + 1, kvvb, qvb, semgb)
            wait_gathers(n, kvva, qva, semga)
            _, dvn = idxvecs(n)

            @pl.when(i2 > 0)
            def _wsa():
                pltpu.make_async_copy(ova, acc.at[dvn], semsa).wait()

            compute(qva, kvva, ova)
            pltpu.async_copy(ova, acc.at[dvn], semsa, add=True)

            @pl.when(i2 < PAIRS - 1)
            def _prefetch_a():
                start_gathers(n + 2, kvva, qva, semga)

            wait_gathers(n + 1, kvvb, qvb, semgb)
            _, dvn1 = idxvecs(n + 1)

            @pl.when(i2 > 0)
            def _wsb():
                pltpu.make_async_copy(ovb, acc.at[dvn1], semsb).wait()

            compute(qvb, kvvb, ovb)
            pltpu.async_copy(ovb, acc.at[dvn1], semsb, add=True)
            return c2

        lax.fori_loop(0, PAIRS, pair, 0)
        zv = jnp.zeros((CP,), jnp.int32)
        pltpu.make_async_copy(ova, acc.at[zv], semsa).wait()
        pltpu.make_async_copy(ovb, acc.at[zv], semsb).wait()
        return carry

    lax.fori_loop(0, NSEG, seg_body, 0)
    plsc.subcore_barrier()

    pltpu.sync_copy(acc.at[pl.ds(row0, RPT)], out_ref.at[cid, pl.ds(row0, RPT)])

    @pl.when(sid == NS - 1)
    def _flush_tail():
        pltpu.sync_copy(acc.at[pl.ds(NS * RPT, TAIL)],
                        out_ref.at[cid, pl.ds(NS * RPT, TAIL)])


def _edge_attn(edge_index, qt, kvt):
    mesh = plsc.VectorSubcoreMesh(core_axis_name="c", subcore_axis_name="s")
    f = functools.partial(
        pl.kernel,
        out_type=jax.ShapeDtypeStruct((NC, N, DIM), jnp.float32),
        mesh=mesh,
        scratch_types=[
            pltpu.VMEM((SEGE,), jnp.int32),
            pltpu.VMEM((SEGE,), jnp.int32),
            pltpu.VMEM((CP, DIM), jnp.float32),
            pltpu.VMEM((CP, DIM), jnp.float32),
            pltpu.VMEM((CP, DIM), jnp.float32),
            pltpu.VMEM((CP, DIM), jnp.float32),
            pltpu.VMEM((CP, DIM), jnp.float32),
            pltpu.VMEM((CP, DIM), jnp.float32),
            pltpu.VMEM((ZROWS, DIM), jnp.float32),
            pltpu.VMEM_SHARED((N, DIM), jnp.float32),
            pltpu.SemaphoreType.DMA,
            pltpu.SemaphoreType.DMA,
            pltpu.SemaphoreType.DMA,
            pltpu.SemaphoreType.DMA,
        ],
        compiler_params=pltpu.CompilerParams(needs_layout_passes=False),
    )(_edge_body)
    return f(edge_index[0], edge_index[1], qt, kvt)


def kernel(h_c, h_p, edge_index, Wq_c, Wq_p, Wk, Wv):
    qt, kvt = _project(h_c, h_p, Wq_c, Wq_p, Wk, Wv)
    o2 = _edge_attn(edge_index, qt, kvt)
    out_c = jnp.concatenate([o2[0, :, :HCOLS], o2[1, :, :HCOLS]], axis=1)
    out_p = jnp.concatenate([o2[0, :, HCOLS:], o2[1, :, HCOLS:]], axis=1)
    return out_c, out_p


# speedup vs baseline: 1.0012x; 1.0012x over previous
import functools

import jax
import jax.numpy as jnp
from jax import lax
from jax.experimental import pallas as pl
from jax.experimental.pallas import tpu as pltpu
from jax.experimental.pallas import tpu_sc as plsc

N = 10000
E = 320000
IN_DIM = 128
DIM = 128
NUM_HEAD = 8
HEAD_DIM = DIM // NUM_HEAD
SCALE = 1.0 / DIM ** 0.5

NC = 2
NS = 16
HGRP = 4
HCOLS = HGRP * HEAD_DIM

EPS = E // NS
CP = 16
SEGE = 4000
NSEG = EPS // SEGE
NCHS = SEGE // CP
PAIRS = NCHS // 2
RPT = 624
ZROWS = 16
TAIL = N - NS * RPT


def _proj_body(hc_ref, hp_ref, wqc_ref, wqp_ref, wk_ref, wv_ref,
               qt_ref, kvt_ref):
    hc = hc_ref[...]
    hp = hp_ref[...]
    qc = jnp.dot(hc, wqc_ref[...], preferred_element_type=jnp.float32) * SCALE
    qp = jnp.dot(hp, wqp_ref[...], preferred_element_type=jnp.float32) * SCALE
    k = jnp.dot(hc, wk_ref[...], preferred_element_type=jnp.float32)
    v = jnp.dot(hc, wv_ref[...], preferred_element_type=jnp.float32)
    qt_ref[0] = jnp.concatenate([qc[:, :HCOLS], qp[:, :HCOLS]], axis=1)
    qt_ref[1] = jnp.concatenate([qc[:, HCOLS:], qp[:, HCOLS:]], axis=1)
    kvt_ref[0] = jnp.concatenate([k[:, :HCOLS], v[:, :HCOLS]], axis=1)
    kvt_ref[1] = jnp.concatenate([k[:, HCOLS:], v[:, HCOLS:]], axis=1)


def _project(h_c, h_p, Wq_c, Wq_p, Wk, Wv):
    R = 1000
    grid = (N // R,)
    in_specs = [
        pl.BlockSpec((R, IN_DIM), lambda i: (i, 0)),
        pl.BlockSpec((R, IN_DIM), lambda i: (i, 0)),
    ] + [pl.BlockSpec((IN_DIM, DIM), lambda i: (0, 0))] * 4
    out_specs = [pl.BlockSpec((NC, R, DIM), lambda i: (0, i, 0))] * 2
    out_shape = [jax.ShapeDtypeStruct((NC, N, DIM), jnp.float32)] * 2
    return pl.pallas_call(
        _proj_body,
        grid=grid,
        in_specs=in_specs,
        out_specs=out_specs,
        out_shape=out_shape,
    )(h_c, h_p, Wq_c, Wq_p, Wk, Wv)


def _edge_body(src_ref, dst_ref, qt_ref, kvt_ref, out_ref,
               src_seg, dst_seg,
               sga, dga, sgb, dgb, dsa, dsb,
               qva, kvva, qvb, kvvb, ova, ovb, zb, acc,
               semga, semgb, semsa, semsb):
    cid = lax.axis_index("c")
    sid = lax.axis_index("s")

    zeros16 = jnp.zeros((16,), jnp.float32)

    def zrow(r, carry):
        for j in range(DIM // 16):
            zb[r, pl.ds(j * 16, 16)] = zeros16
        return carry

    lax.fori_loop(0, ZROWS, zrow, 0)
    row0 = sid * RPT
    for j in range(RPT // ZROWS):
        pltpu.sync_copy(zb, acc.at[pl.ds(row0 + j * ZROWS, ZROWS)])

    @pl.when(sid == NS - 1)
    def _zero_tail():
        pltpu.sync_copy(zb.at[pl.ds(0, TAIL)], acc.at[pl.ds(NS * RPT, TAIL)])

    plsc.subcore_barrier()

    qt = qt_ref.at[cid]
    kvt = kvt_ref.at[cid]

    rows = lax.iota(jnp.int32, 16)

    def start_gathers(ci, kvb, qb, sem, sref, dref):
        sref[...] = src_seg[pl.ds(ci * CP, CP)]
        dref[...] = dst_seg[pl.ds(ci * CP, CP)]
        pltpu.async_copy(kvt.at[sref], kvb, sem)
        pltpu.async_copy(qt.at[dref], qb, sem)

    def wait_gathers(kvb, qb, sem, sref, dref):
        pltpu.make_async_copy(kvt.at[sref], kvb, sem).wait()
        pltpu.make_async_copy(qt.at[dref], qb, sem).wait()

    def compute(qb, kvb, ov):
        for h in range(HGRP):
            colv = jnp.full((16,), h * HEAD_DIM, jnp.int32)
            pc = []
            pp = []
            for d in range(HEAD_DIM):
                cv = colv + d
                cv64 = cv + HCOLS
                kc = plsc.load_gather(kvb, [rows, cv])
                qcc = plsc.load_gather(qb, [rows, cv])
                qpc = plsc.load_gather(qb, [rows, cv64])
                pc.append(qcc * kc)
                pp.append(qpc * kc)
            while len(pc) > 1:
                pc = [pc[i] + pc[i + 1] for i in range(0, len(pc), 2)]
                pp = [pp[i] + pp[i + 1] for i in range(0, len(pp), 2)]
            wc = jnp.exp(jnp.clip(pc[0], -5.0, 5.0))
            wp = jnp.exp(jnp.clip(pp[0], -5.0, 5.0))
            for d in range(HEAD_DIM):
                cv = colv + d
                cv64 = cv + HCOLS
                vc = plsc.load_gather(kvb, [rows, cv64])
                plsc.store_scatter(ov, [rows, cv], vc * wc)
                plsc.store_scatter(ov, [rows, cv64], vc * wp)

    def seg_body(sg, carry):
        segbase = sid * EPS + sg * SEGE
        pltpu.sync_copy(src_ref.at[pl.ds(segbase, SEGE)], src_seg)
        pltpu.sync_copy(dst_ref.at[pl.ds(segbase, SEGE)], dst_seg)

        start_gathers(0, kvva, qva, semga, sga, dga)

        def pair(i2, c2):
            n = 2 * i2
            start_gathers(n + 1, kvvb, qvb, semgb, sgb, dgb)
            wait_gathers(kvva, qva, semga, sga, dga)

            @pl.when(i2 > 0)
            def _wsa():
                pltpu.make_async_copy(ova, acc.at[dsa], semsa).wait()

            dsa[...] = dga[...]
            compute(qva, kvva, ova)
            pltpu.async_copy(ova, acc.at[dsa], semsa, add=True)

            @pl.when(i2 < PAIRS - 1)
            def _prefetch_a():
                start_gathers(n + 2, kvva, qva, semga, sga, dga)

            wait_gathers(kvvb, qvb, semgb, sgb, dgb)

            @pl.when(i2 > 0)
            def _wsb():
                pltpu.make_async_copy(ovb, acc.at[dsb], semsb).wait()

            dsb[...] = dgb[...]
            compute(qvb, kvvb, ovb)
            pltpu.async_copy(ovb, acc.at[dsb], semsb, add=True)
            return c2

        lax.fori_loop(0, PAIRS, pair, 0)
        pltpu.make_async_copy(ova, acc.at[dsa], semsa).wait()
        pltpu.make_async_copy(ovb, acc.at[dsb], semsb).wait()
        return carry

    lax.fori_loop(0, NSEG, seg_body, 0)
    plsc.subcore_barrier()

    pltpu.sync_copy(acc.at[pl.ds(row0, RPT)], out_ref.at[cid, pl.ds(row0, RPT)])

    @pl.when(sid == NS - 1)
    def _flush_tail():
        pltpu.sync_copy(acc.at[pl.ds(NS * RPT, TAIL)],
                        out_ref.at[cid, pl.ds(NS * RPT, TAIL)])


def _edge_attn(edge_index, qt, kvt):
    mesh = plsc.VectorSubcoreMesh(core_axis_name="c", subcore_axis_name="s")
    f = functools.partial(
        pl.kernel,
        out_type=jax.ShapeDtypeStruct((NC, N, DIM), jnp.float32),
        mesh=mesh,
        scratch_types=[
            pltpu.VMEM((SEGE,), jnp.int32),
            pltpu.VMEM((SEGE,), jnp.int32),
            pltpu.VMEM((CP,), jnp.int32),
            pltpu.VMEM((CP,), jnp.int32),
            pltpu.VMEM((CP,), jnp.int32),
            pltpu.VMEM((CP,), jnp.int32),
            pltpu.VMEM((CP,), jnp.int32),
            pltpu.VMEM((CP,), jnp.int32),
            pltpu.VMEM((CP, DIM), jnp.float32),
            pltpu.VMEM((CP, DIM), jnp.float32),
            pltpu.VMEM((CP, DIM), jnp.float32),
            pltpu.VMEM((CP, DIM), jnp.float32),
            pltpu.VMEM((CP, DIM), jnp.float32),
            pltpu.VMEM((CP, DIM), jnp.float32),
            pltpu.VMEM((ZROWS, DIM), jnp.float32),
            pltpu.VMEM_SHARED((N, DIM), jnp.float32),
            pltpu.SemaphoreType.DMA,
            pltpu.SemaphoreType.DMA,
            pltpu.SemaphoreType.DMA,
            pltpu.SemaphoreType.DMA,
        ],
        compiler_params=pltpu.CompilerParams(needs_layout_passes=False),
    )(_edge_body)
    return f(edge_index[0], edge_index[1], qt, kvt)


def kernel(h_c, h_p, edge_index, Wq_c, Wq_p, Wk, Wv):
    qt, kvt = _project(h_c, h_p, Wq_c, Wq_p, Wk, Wv)
    o2 = _edge_attn(edge_index, qt, kvt)
    out_c = jnp.concatenate([o2[0, :, :HCOLS], o2[1, :, :HCOLS]], axis=1)
    out_p = jnp.concatenate([o2[0, :, HCOLS:], o2[1, :, HCOLS:]], axis=1)
    return out_c, out_p


# speedup vs baseline: 7.1557x; 7.1472x over previous
import functools

import jax
import jax.numpy as jnp
from jax import lax
from jax.experimental import pallas as pl
from jax.experimental.pallas import tpu as pltpu
from jax.experimental.pallas import tpu_sc as plsc

N = 10000
E = 320000
IN_DIM = 128
DIM = 128
NUM_HEAD = 8
HEAD_DIM = DIM // NUM_HEAD
SCALE = 1.0 / DIM ** 0.5

NC = 2
NS = 16
HGRP = 4
HCOLS = HGRP * HEAD_DIM

EPS = E // NS
CP = 50
SEGE = 400
NSEG = EPS // SEGE
NCHS = SEGE // CP
PAIRS = NCHS // 2
EROWS = E // CP
RPT = 624
TAIL = N - NS * RPT


def _proj_body(hc_ref, hp_ref, wqc_ref, wqp_ref, wk_ref, wv_ref,
               qt_ref, kvt_ref):
    hc = hc_ref[...]
    hp = hp_ref[...]
    qc = jnp.dot(hc, wqc_ref[...], preferred_element_type=jnp.float32) * SCALE
    qp = jnp.dot(hp, wqp_ref[...], preferred_element_type=jnp.float32) * SCALE
    k = jnp.dot(hc, wk_ref[...], preferred_element_type=jnp.float32)
    v = jnp.dot(hc, wv_ref[...], preferred_element_type=jnp.float32)
    qt_ref[0] = jnp.concatenate([qc[:, :HCOLS], qp[:, :HCOLS]], axis=1)
    qt_ref[1] = jnp.concatenate([qc[:, HCOLS:], qp[:, HCOLS:]], axis=1)
    kvt_ref[0] = jnp.concatenate([k[:, :HCOLS], v[:, :HCOLS]], axis=1)
    kvt_ref[1] = jnp.concatenate([k[:, HCOLS:], v[:, HCOLS:]], axis=1)


def _project(h_c, h_p, Wq_c, Wq_p, Wk, Wv):
    R = 1000
    grid = (N // R,)
    in_specs = [
        pl.BlockSpec((R, IN_DIM), lambda i: (i, 0)),
        pl.BlockSpec((R, IN_DIM), lambda i: (i, 0)),
    ] + [pl.BlockSpec((IN_DIM, DIM), lambda i: (0, 0))] * 4
    out_specs = [pl.BlockSpec((NC, R, DIM), lambda i: (0, i, 0))] * 2
    out_shape = [jax.ShapeDtypeStruct((NC, N, DIM), jnp.float32)] * 2
    return pl.pallas_call(
        _proj_body,
        grid=grid,
        in_specs=in_specs,
        out_specs=out_specs,
        out_shape=out_shape,
    )(h_c, h_p, Wq_c, Wq_p, Wk, Wv)


def _edge_body(src_ref, dst_ref, qt_ref, kvt_ref, out_ref,
               src_seg, dst_seg,
               qva, kvva, qvb, kvvb, ova, ovb, acc,
               semga, semgb, semsa, semsb):
    cid = lax.axis_index("c")
    sid = lax.axis_index("s")

    zeros16 = jnp.zeros((16,), jnp.float32)

    def zrow(r, carry):
        for j in range(DIM // 16):
            ova[r, pl.ds(j * 16, 16)] = zeros16
        return carry

    lax.fori_loop(0, CP, zrow, 0)
    row0 = sid * RPT
    for j in range(RPT // CP):
        pltpu.sync_copy(ova, acc.at[pl.ds(row0 + j * CP, CP)])
    pltpu.sync_copy(ova.at[pl.ds(0, RPT % CP)],
                    acc.at[pl.ds(row0 + (RPT // CP) * CP, RPT % CP)])

    @pl.when(sid == NS - 1)
    def _zero_tail():
        pltpu.sync_copy(ova.at[pl.ds(0, TAIL)], acc.at[pl.ds(NS * RPT, TAIL)])

    plsc.subcore_barrier()

    qt = qt_ref.at[cid]
    kvt = kvt_ref.at[cid]

    lanes = lax.iota(jnp.int32, 16)
    pidx = [lanes ^ 8, lanes ^ 4, lanes ^ 2, lanes ^ 1]
    fmask = [(lanes & 8) == 0, (lanes & 4) == 0, (lanes & 2) == 0]
    lane_of = (0, 8, 4, 12, 2, 10, 6, 14)
    bcast = [jnp.full((16,), lane_of[j], jnp.int32) for j in range(8)]
    dnums = lax.GatherDimensionNumbers(
        offset_dims=(), collapsed_slice_dims=(0,), start_index_map=(0,))

    def take(x, p):
        return lax.gather(x, p[:, None], dnums, (1,),
                          mode=lax.GatherScatterMode.PROMISE_IN_BOUNDS)

    def start_gathers(ci, kvb, qb, sem):
        pltpu.async_copy(kvt.at[src_seg.at[ci]], kvb, sem)
        pltpu.async_copy(qt.at[dst_seg.at[ci]], qb, sem)

    def wait_gathers(ci, kvb, qb, sem):
        pltpu.make_async_copy(kvt.at[src_seg.at[ci]], kvb, sem).wait()
        pltpu.make_async_copy(qt.at[dst_seg.at[ci]], qb, sem).wait()

    def fold(a, b, pv, m):
        return jnp.where(m, a + take(a, pv), b + take(b, pv))

    def compute(qb, kvb, ov):
        def edge_one(e, inner):
            prods = []
            vvs = []
            for h in range(HGRP):
                sl = pl.ds(h * HEAD_DIM, HEAD_DIM)
                slv = pl.ds(HCOLS + h * HEAD_DIM, HEAD_DIM)
                kk = kvb[e, sl]
                vvs.append(kvb[e, slv])
                prods.append(qb[e, sl] * kk)
                prods.append(qb[e, slv] * kk)
            v1 = [fold(prods[2 * i], prods[2 * i + 1], pidx[0], fmask[0])
                  for i in range(4)]
            v2 = [fold(v1[2 * i], v1[2 * i + 1], pidx[1], fmask[1])
                  for i in range(2)]
            x = fold(v2[0], v2[1], pidx[2], fmask[2])
            x = x + take(x, pidx[3])
            w = jnp.exp(jnp.clip(x, -5.0, 5.0))
            for h in range(HGRP):
                sl = pl.ds(h * HEAD_DIM, HEAD_DIM)
                slv = pl.ds(HCOLS + h * HEAD_DIM, HEAD_DIM)
                ov[e, sl] = vvs[h] * take(w, bcast[2 * h])
                ov[e, slv] = vvs[h] * take(w, bcast[2 * h + 1])
            return inner

        lax.fori_loop(0, CP, edge_one, 0)

    def seg_body(sg, carry):
        rowbase = sid * (EPS // CP) + sg * NCHS
        pltpu.sync_copy(src_ref.at[pl.ds(rowbase, NCHS)], src_seg)
        pltpu.sync_copy(dst_ref.at[pl.ds(rowbase, NCHS)], dst_seg)

        start_gathers(0, kvva, qva, semga)

        def pair(i2, c2):
            n = 2 * i2
            start_gathers(n + 1, kvvb, qvb, semgb)
            wait_gathers(n, kvva, qva, semga)

            @pl.when(i2 > 0)
            def _wsa():
                pltpu.make_async_copy(ova, acc.at[dst_seg.at[0]], semsa).wait()

            compute(qva, kvva, ova)
            pltpu.async_copy(ova, acc.at[dst_seg.at[n]], semsa, add=True)

            @pl.when(i2 < PAIRS - 1)
            def _prefetch_a():
                start_gathers(n + 2, kvva, qva, semga)

            wait_gathers(n + 1, kvvb, qvb, semgb)

            @pl.when(i2 > 0)
            def _wsb():
                pltpu.make_async_copy(ovb, acc.at[dst_seg.at[0]], semsb).wait()

            compute(qvb, kvvb, ovb)
            pltpu.async_copy(ovb, acc.at[dst_seg.at[n + 1]], semsb, add=True)
            return c2

        lax.fori_loop(0, PAIRS, pair, 0)
        pltpu.make_async_copy(ova, acc.at[dst_seg.at[0]], semsa).wait()
        pltpu.make_async_copy(ovb, acc.at[dst_seg.at[0]], semsb).wait()
        return carry

    lax.fori_loop(0, NSEG, seg_body, 0)
    plsc.subcore_barrier()

    pltpu.sync_copy(acc.at[pl.ds(row0, RPT)], out_ref.at[cid, pl.ds(row0, RPT)])

    @pl.when(sid == NS - 1)
    def _flush_tail():
        pltpu.sync_copy(acc.at[pl.ds(NS * RPT, TAIL)],
                        out_ref.at[cid, pl.ds(NS * RPT, TAIL)])


def _edge_attn(edge_index, qt, kvt):
    mesh = plsc.VectorSubcoreMesh(core_axis_name="c", subcore_axis_name="s")
    f = functools.partial(
        pl.kernel,
        out_type=jax.ShapeDtypeStruct((NC, N, DIM), jnp.float32),
        mesh=mesh,
        scratch_types=[
            pltpu.VMEM((NCHS, CP), jnp.int32),
            pltpu.VMEM((NCHS, CP), jnp.int32),
            pltpu.VMEM((CP, DIM), jnp.float32),
            pltpu.VMEM((CP, DIM), jnp.float32),
            pltpu.VMEM((CP, DIM), jnp.float32),
            pltpu.VMEM((CP, DIM), jnp.float32),
            pltpu.VMEM((CP, DIM), jnp.float32),
            pltpu.VMEM((CP, DIM), jnp.float32),
            pltpu.VMEM_SHARED((N, DIM), jnp.float32),
            pltpu.SemaphoreType.DMA,
            pltpu.SemaphoreType.DMA,
            pltpu.SemaphoreType.DMA,
            pltpu.SemaphoreType.DMA,
        ],
        compiler_params=pltpu.CompilerParams(needs_layout_passes=False),
    )(_edge_body)
    return f(edge_index[0].reshape(EROWS, CP), edge_index[1].reshape(EROWS, CP),
             qt, kvt)


def kernel(h_c, h_p, edge_index, Wq_c, Wq_p, Wk, Wv):
    qt, kvt = _project(h_c, h_p, Wq_c, Wq_p, Wk, Wv)
    o2 = _edge_attn(edge_index, qt, kvt)
    out_c = jnp.concatenate([o2[0, :, :HCOLS], o2[1, :, :HCOLS]], axis=1)
    out_p = jnp.concatenate([o2[0, :, HCOLS:], o2[1, :, HCOLS:]], axis=1)
    return out_c, out_p
